# Initial kernel scaffold; baseline (speedup 1.0000x reference)
import jax
import jax.numpy as jnp
from jax.experimental import pallas as pl


def kernel(x, W_def, b_def):
    raise NotImplementedError("write your pallas kernel here")


# speedup vs baseline: 19.1771x; 19.1771x over previous
import functools

import jax
import jax.numpy as jnp
from jax.experimental import pallas as pl
from jax.experimental.pallas import tpu as pltpu

_HB = 8


def _body(x_ref, coef_ref, gx_ref, ylin_ref, b_ref, out_ref):
    gx = gx_ref[...]
    gz = coef_ref[15:16, :]
    b0 = b_ref[0, 0]
    b1 = b_ref[0, 1]
    b2 = b_ref[0, 2]
    h_base = pl.program_id(1) * _HB
    for hh in range(_HB):
        X = x_ref[0, hh]
        X = X.astype(jnp.bfloat16).astype(jnp.float32)
        t = [jnp.full_like(X, b) for b in (b0, b1, b2)]
        for si, s in enumerate((-2, -1, 0, 1, 2)):
            Xs = X if s == 0 else jnp.roll(X, -s, axis=1)
            for o in range(3):
                t[o] = t[o] + coef_ref[si * 3 + o:si * 3 + o + 1, :] * Xs
        gy = ylin_ref[0, h_base + hh]
        xf = (t[0] * gx + 1.0) * 32.0
        yf = (t[1] * gy + 1.0) * 16.0
        zf = (t[2] * gz + 1.0) * 32.0
        x0 = jnp.floor(xf)
        y0 = jnp.floor(yf)
        z0 = jnp.floor(zf)
        x0c = jnp.clip(x0, 0.0, 63.0)
        x1c = jnp.clip(x0 + 1.0, 0.0, 63.0)
        y0c = jnp.clip(y0, 0.0, 31.0)
        y1c = jnp.clip(y0 + 1.0, 0.0, 31.0)
        z0c = jnp.clip(z0, 0.0, 63.0)
        z1c = jnp.clip(z0 + 1.0, 0.0, 63.0)
        ry0 = (y0c - 8.0) ** 2
        ry1 = (y1c - 8.0) ** 2
        rx0 = (x0c - 10.0) ** 2
        rx1 = (x1c - 10.0) ** 2
        rz0 = (z0c - 10.0) ** 2
        rz1 = (z1c - 10.0) ** 2
        dx = x1c - xf
        dy = y1c - yf
        dz = z1c - zf
        exdx = 1.0 - dx
        exdy = 1.0 - dy
        exdz = 1.0 - dz
        acc = None
        for ry, py in ((ry0, dy), (ry1, exdy)):
            for rx, px in ((rx0, dx), (rx1, exdx)):
                sxy = ry + rx
                pxy = py * px
                for rz, pz in ((rz0, dz), (rz1, exdz)):
                    term = jnp.where(sxy + rz <= 121.0, pxy * pz, 0.0)
                    acc = term if acc is None else acc + term
        out_ref[0, hh] = acc


@jax.jit
def kernel(x, W_def, b_def):
    B, H, W, D, C = x.shape
    x4 = x.reshape(B, H, W, D * C)
    p = jnp.arange(192) % 3
    rows = []
    for s in (-2, -1, 0, 1, 2):
        idx = p + s
        valid = (idx >= 0) & (idx < 3)
        idxc = jnp.clip(idx, 0, 2)
        for o in range(3):
            rows.append(jnp.where(valid, W_def[idxc, o], 0.0))
    z_lin = jnp.linspace(-1.0, 1.0, D)
    rows.append(z_lin[jnp.arange(192) // 3])
    coef = jnp.stack(rows, axis=0).astype(jnp.float32)
    coef = coef.at[:15].set(
        coef[:15].astype(jnp.bfloat16).astype(jnp.float32))
    x_lin = jnp.linspace(-1.0, 1.0, W)
    gx = jnp.broadcast_to(x_lin[:, None], (W, 192)).astype(jnp.float32)
    ylin = jnp.linspace(-1.0, 1.0, H).reshape(1, H).astype(jnp.float32)
    bvec = b_def.reshape(1, 3).astype(jnp.float32)

    out4 = pl.pallas_call(
        _body,
        grid=(B, H // _HB),
        in_specs=[
            pl.BlockSpec((1, _HB, W, 192), lambda b, h: (b, h, 0, 0)),
            pl.BlockSpec((16, 192), lambda b, h: (0, 0)),
            pl.BlockSpec((W, 192), lambda b, h: (0, 0)),
            pl.BlockSpec(memory_space=pltpu.SMEM),
            pl.BlockSpec(memory_space=pltpu.SMEM),
        ],
        out_specs=pl.BlockSpec((1, _HB, W, 192), lambda b, h: (b, h, 0, 0)),
        out_shape=jax.ShapeDtypeStruct((B, H, W, 192), jnp.float32),
        compiler_params=pltpu.CompilerParams(
            dimension_semantics=("parallel", "parallel"),
        ),
    )(x4, coef, gx, ylin, bvec)
    return out4.reshape(B, H, W, D, C)


# speedup vs baseline: 22.4666x; 1.1715x over previous
import jax
import jax.numpy as jnp
from jax.experimental import pallas as pl
from jax.experimental.pallas import tpu as pltpu

_HB = 8


def _body(x_ref, m_ref, gx_ref, ylin_ref, b_ref, out_ref):
    gx = gx_ref[0:64, :]
    gz = gx_ref[64:65, :]
    b0 = b_ref[0, 0]
    b1 = b_ref[0, 1]
    b2 = b_ref[0, 2]
    h_base = pl.program_id(1) * _HB
    X = x_ref[0].reshape(_HB * 64, 192).astype(jnp.bfloat16)
    T = jax.lax.dot_general(
        X, m_ref[...], (((1,), (0,)), ((), ())),
        preferred_element_type=jnp.float32)
    for hh in range(_HB):
        r0 = hh * 64
        t0 = T[r0:r0 + 64, 0:192] + b0
        t1 = T[r0:r0 + 64, 256:448] + b1
        t2 = T[r0:r0 + 64, 512:704] + b2
        gy = ylin_ref[0, h_base + hh]
        xf = (t0 * gx + 1.0) * 32.0
        yf = (t1 * gy + 1.0) * 16.0
        zf = (t2 * gz + 1.0) * 32.0
        x0 = jnp.floor(xf)
        y0 = jnp.floor(yf)
        z0 = jnp.floor(zf)
        x0c = jnp.clip(x0, 0.0, 63.0)
        x1c = jnp.clip(x0 + 1.0, 0.0, 63.0)
        y0c = jnp.clip(y0, 0.0, 31.0)
        y1c = jnp.clip(y0 + 1.0, 0.0, 31.0)
        z0c = jnp.clip(z0, 0.0, 63.0)
        z1c = jnp.clip(z0 + 1.0, 0.0, 63.0)
        ry0 = (y0c - 8.0) ** 2
        ry1 = (y1c - 8.0) ** 2
        rx0 = (x0c - 10.0) ** 2
        rx1 = (x1c - 10.0) ** 2
        rz0 = (z0c - 10.0) ** 2
        rz1 = (z1c - 10.0) ** 2
        dx = x1c - xf
        dy = y1c - yf
        dz = z1c - zf
        exdx = 1.0 - dx
        exdy = 1.0 - dy
        exdz = 1.0 - dz
        acc = None
        for ry, py in ((ry0, dy), (ry1, exdy)):
            for rx, px in ((rx0, dx), (rx1, exdx)):
                u = 121.0 - (ry + rx)
                pxy = py * px
                q = jnp.where(rz0 <= u, dz, 0.0) + jnp.where(rz1 <= u, exdz, 0.0)
                term = pxy * q
                acc = term if acc is None else acc + term
        out_ref[0, hh] = acc


@jax.jit
def kernel(x, W_def, b_def):
    B, H, W, D, C = x.shape
    x4 = x.reshape(B, H, W, D * C)
    l = jnp.arange(192)
    d = l // 3
    Wb = W_def.astype(jnp.bfloat16)
    cols = []
    for o in range(3):
        blk = jnp.where(l[:, None] // 3 == d[None, :], Wb[l % 3, o][:, None],
                        jnp.bfloat16(0))
        blk = jnp.pad(blk, ((0, 0), (0, 64)))
        cols.append(blk)
    M = jnp.concatenate(cols, axis=1)
    z_lin = jnp.linspace(-1.0, 1.0, D)
    x_lin = jnp.linspace(-1.0, 1.0, W)
    gx = jnp.broadcast_to(x_lin[:, None], (W, 192)).astype(jnp.float32)
    gxz = jnp.concatenate([gx, z_lin[d].reshape(1, 192)], axis=0)
    gxz = jnp.pad(gxz, ((0, 7), (0, 0)))
    ylin = jnp.linspace(-1.0, 1.0, H).reshape(1, H).astype(jnp.float32)
    bvec = b_def.reshape(1, 3).astype(jnp.float32)

    out4 = pl.pallas_call(
        _body,
        grid=(B, H // _HB),
        in_specs=[
            pl.BlockSpec((1, _HB, W, 192), lambda b, h: (b, h, 0, 0)),
            pl.BlockSpec((192, 768), lambda b, h: (0, 0)),
            pl.BlockSpec((72, 192), lambda b, h: (0, 0)),
            pl.BlockSpec(memory_space=pltpu.SMEM),
            pl.BlockSpec(memory_space=pltpu.SMEM),
        ],
        out_specs=pl.BlockSpec((1, _HB, W, 192), lambda b, h: (b, h, 0, 0)),
        out_shape=jax.ShapeDtypeStruct((B, H, W, 192), jnp.float32),
        compiler_params=pltpu.CompilerParams(
            dimension_semantics=("parallel", "parallel"),
        ),
    )(x4, M, gxz, ylin, bvec)
    return out4.reshape(B, H, W, D, C)


# speedup vs baseline: 22.9553x; 1.0218x over previous
import jax
import jax.numpy as jnp
from jax.experimental import pallas as pl
from jax.experimental.pallas import tpu as pltpu

_HB = 8


def _body(x_ref, m_ref, c_ref, sy_ref, out_ref):
    gx32 = c_ref[0:64, :]
    hx = c_ref[64:128, :]
    gz32 = c_ref[128:129, :]
    hz = c_ref[129:130, :]
    h_base = pl.program_id(1) * _HB
    X = x_ref[0].reshape(_HB * 64, 192).astype(jnp.bfloat16)
    T = jax.lax.dot_general(
        X, m_ref[...], (((1,), (0,)), ((), ())),
        preferred_element_type=jnp.float32)
    for hh in range(_HB):
        r0 = hh * 64
        t0 = T[r0:r0 + 64, 0:192]
        t1 = T[r0:r0 + 64, 256:448]
        t2 = T[r0:r0 + 64, 512:704]
        sy = sy_ref[0, h_base + hh]
        cy = sy_ref[1, h_base + hh]
        xf = t0 * gx32 + hx
        yf = t1 * sy + cy
        zf = t2 * gz32 + hz
        x0 = jnp.floor(xf)
        y0 = jnp.floor(yf)
        z0 = jnp.floor(zf)
        x0c = jnp.clip(x0, 0.0, 63.0)
        x1c = jnp.clip(x0 + 1.0, 0.0, 63.0)
        y0c = jnp.clip(y0, 0.0, 31.0)
        y1c = jnp.clip(y0 + 1.0, 0.0, 31.0)
        z0c = jnp.clip(z0, 0.0, 63.0)
        z1c = jnp.clip(z0 + 1.0, 0.0, 63.0)
        ya0 = y0c - 8.0
        ya1 = y1c - 8.0
        xa0 = x0c - 10.0
        xa1 = x1c - 10.0
        za0 = z0c - 10.0
        za1 = z1c - 10.0
        uy0 = 121.0 - ya0 * ya0
        uy1 = 121.0 - ya1 * ya1
        rx0 = xa0 * xa0
        rx1 = xa1 * xa1
        rz0 = za0 * za0
        rz1 = za1 * za1
        dx = x1c - xf
        dy = y1c - yf
        dz = z1c - zf
        exdx = 1.0 - dx
        exdy = 1.0 - dy
        exdz = 1.0 - dz
        acc = None
        for uy, py in ((uy0, dy), (uy1, exdy)):
            for rx, px in ((rx0, dx), (rx1, exdx)):
                u = uy - rx
                pxy = py * px
                q = jnp.where(rz0 <= u, dz, 0.0) + jnp.where(rz1 <= u, exdz, 0.0)
                term = pxy * q
                acc = term if acc is None else acc + term
        out_ref[0, hh] = acc


@jax.jit
def kernel(x, W_def, b_def):
    B, H, W, D, C = x.shape
    x4 = x.reshape(B, H, W, D * C)
    l = jnp.arange(192)
    d = l // 3
    Wb = W_def.astype(jnp.bfloat16)
    cols = []
    for o in range(3):
        blk = jnp.where(l[:, None] // 3 == d[None, :], Wb[l % 3, o][:, None],
                        jnp.bfloat16(0))
        blk = jnp.pad(blk, ((0, 0), (0, 64)))
        cols.append(blk)
    M = jnp.concatenate(cols, axis=1)
    b0, b1, b2 = b_def[0], b_def[1], b_def[2]
    z_lin = jnp.linspace(-1.0, 1.0, D)
    x_lin = jnp.linspace(-1.0, 1.0, W)
    y_lin = jnp.linspace(-1.0, 1.0, H)
    gx32 = jnp.broadcast_to(32.0 * x_lin[:, None], (W, 192))
    hx = b0 * gx32 + 32.0
    gz32 = (32.0 * z_lin[d]).reshape(1, 192)
    hz = b2 * gz32 + 32.0
    consts = jnp.concatenate(
        [gx32, hx, gz32, hz], axis=0).astype(jnp.float32)
    consts = jnp.pad(consts, ((0, 6), (0, 0)))
    sy = 16.0 * y_lin
    cy = b1 * sy + 16.0
    sycy = jnp.stack([sy, cy], axis=0).astype(jnp.float32)

    out4 = pl.pallas_call(
        _body,
        grid=(B, H // _HB),
        in_specs=[
            pl.BlockSpec((1, _HB, W, 192), lambda b, h: (b, h, 0, 0)),
            pl.BlockSpec((192, 768), lambda b, h: (0, 0)),
            pl.BlockSpec((136, 192), lambda b, h: (0, 0)),
            pl.BlockSpec(memory_space=pltpu.SMEM),
        ],
        out_specs=pl.BlockSpec((1, _HB, W, 192), lambda b, h: (b, h, 0, 0)),
        out_shape=jax.ShapeDtypeStruct((B, H, W, 192), jnp.float32),
        compiler_params=pltpu.CompilerParams(
            dimension_semantics=("parallel", "parallel"),
        ),
    )(x4, M, consts, sycy)
    return out4.reshape(B, H, W, D, C)


# speedup vs baseline: 55.9114x; 2.4357x over previous
import jax
import jax.numpy as jnp
from jax.experimental import pallas as pl
from jax.experimental.pallas import tpu as pltpu

_HB = 8


def _body(x_ref, c_ref, sy_ref, w_ref, out_ref):
    gx32 = c_ref[0]
    hx = c_ref[1]
    gz32 = c_ref[2]
    hz = c_ref[3]
    h_base = pl.program_id(1) * _HB
    for hh in range(_HB):
        X = x_ref[0, hh].astype(jnp.float32)

        xc0 = X[0]
        xc1 = X[1]
        xc2 = X[2]
        t0 = xc0 * w_ref[0, 0] + xc1 * w_ref[1, 0] + xc2 * w_ref[2, 0]
        t1 = xc0 * w_ref[0, 1] + xc1 * w_ref[1, 1] + xc2 * w_ref[2, 1]
        t2 = xc0 * w_ref[0, 2] + xc1 * w_ref[1, 2] + xc2 * w_ref[2, 2]
        sy = sy_ref[0, h_base + hh]
        cy = sy_ref[1, h_base + hh]
        xf = t0 * gx32 + hx
        yf = t1 * sy + cy
        zf = t2 * gz32 + hz
        x0 = jnp.floor(xf)
        y0 = jnp.floor(yf)
        z0 = jnp.floor(zf)
        x0c = jnp.clip(x0, 0.0, 63.0)
        x1c = jnp.clip(x0 + 1.0, 0.0, 63.0)
        y0c = jnp.clip(y0, 0.0, 31.0)
        y1c = jnp.clip(y0 + 1.0, 0.0, 31.0)
        z0c = jnp.clip(z0, 0.0, 63.0)
        z1c = jnp.clip(z0 + 1.0, 0.0, 63.0)
        ya0 = y0c - 8.0
        ya1 = y1c - 8.0
        xa0 = x0c - 10.0
        xa1 = x1c - 10.0
        za0 = z0c - 10.0
        za1 = z1c - 10.0
        uy0 = 121.0 - ya0 * ya0
        uy1 = 121.0 - ya1 * ya1
        rx0 = xa0 * xa0
        rx1 = xa1 * xa1
        rz0 = za0 * za0
        rz1 = za1 * za1
        dx = x1c - xf
        dy = y1c - yf
        dz = z1c - zf
        exdx = 1.0 - dx
        exdy = 1.0 - dy
        exdz = 1.0 - dz
        acc = None
        for uy, py in ((uy0, dy), (uy1, exdy)):
            for rx, px in ((rx0, dx), (rx1, exdx)):
                u = uy - rx
                pxy = py * px
                q = jnp.where(rz0 <= u, dz, 0.0) + jnp.where(rz1 <= u, exdz, 0.0)
                term = pxy * q
                acc = term if acc is None else acc + term
        out_ref[0, hh, 0] = acc
        out_ref[0, hh, 1] = acc
        out_ref[0, hh, 2] = acc


@jax.jit
def kernel(x, W_def, b_def):
    B, H, W, D, C = x.shape
    xt = jnp.transpose(x, (0, 1, 4, 2, 3)).astype(jnp.bfloat16)
    b0, b1, b2 = b_def[0], b_def[1], b_def[2]
    z_lin = jnp.linspace(-1.0, 1.0, D)
    x_lin = jnp.linspace(-1.0, 1.0, W)
    y_lin = jnp.linspace(-1.0, 1.0, H)
    gx32 = jnp.broadcast_to(32.0 * x_lin[:, None], (W, D))
    hx = b0 * gx32 + 32.0
    gz32 = jnp.broadcast_to(32.0 * z_lin[None, :], (W, D))
    hz = b2 * gz32 + 32.0
    consts = jnp.stack([gx32, hx, gz32, hz], axis=0).astype(jnp.float32)
    sy = 16.0 * y_lin
    cy = b1 * sy + 16.0
    sycy = jnp.stack([sy, cy], axis=0).astype(jnp.float32)
    wu = jax.lax.bitcast_convert_type(W_def, jnp.uint32)
    wu = (wu + jnp.uint32(0x7FFF) + ((wu >> 16) & jnp.uint32(1))) & jnp.uint32(0xFFFF0000)
    Wr = jax.lax.bitcast_convert_type(wu, jnp.float32)

    out5 = pl.pallas_call(
        _body,
        grid=(B, H // _HB),
        in_specs=[
            pl.BlockSpec((1, _HB, C, W, D), lambda b, h: (b, h, 0, 0, 0)),
            pl.BlockSpec((4, W, D), lambda b, h: (0, 0, 0)),
            pl.BlockSpec(memory_space=pltpu.SMEM),
            pl.BlockSpec(memory_space=pltpu.SMEM),
        ],
        out_specs=pl.BlockSpec((1, _HB, C, W, D), lambda b, h: (b, h, 0, 0, 0)),
        out_shape=jax.ShapeDtypeStruct((B, H, C, W, D), jnp.float32),
        compiler_params=pltpu.CompilerParams(
            dimension_semantics=("parallel", "parallel"),
        ),
    )(xt, consts, sycy, Wr)
    return jnp.transpose(out5, (0, 1, 3, 4, 2))


# speedup vs baseline: 67.0593x; 1.1994x over previous
import jax
import jax.numpy as jnp
from jax.experimental import pallas as pl
from jax.experimental.pallas import tpu as pltpu

_HB = 16


def _body(x_ref, c_ref, sy_ref, w_ref, out_ref):
    gx32 = c_ref[0]
    hx = c_ref[1]
    gz32 = c_ref[2]
    hz = c_ref[3]
    h_base = pl.program_id(1) * _HB
    for hh in range(_HB):
        X = x_ref[0, hh].astype(jnp.float32)

        xc0 = X[0]
        xc1 = X[1]
        xc2 = X[2]
        t0 = xc0 * w_ref[0, 0] + xc1 * w_ref[1, 0] + xc2 * w_ref[2, 0]
        t1 = xc0 * w_ref[0, 1] + xc1 * w_ref[1, 1] + xc2 * w_ref[2, 1]
        t2 = xc0 * w_ref[0, 2] + xc1 * w_ref[1, 2] + xc2 * w_ref[2, 2]
        sy = sy_ref[0, h_base + hh]
        cy = sy_ref[1, h_base + hh]
        xf = t0 * gx32 + hx
        yf = t1 * sy + cy
        zf = t2 * gz32 + hz
        x0 = jnp.floor(xf)
        y0 = jnp.floor(yf)
        z0 = jnp.floor(zf)
        x0c = jnp.clip(x0, 0.0, 63.0)
        x1c = jnp.clip(x0 + 1.0, 0.0, 63.0)
        y0c = jnp.clip(y0, 0.0, 31.0)
        y1c = jnp.clip(y0 + 1.0, 0.0, 31.0)
        z0c = jnp.clip(z0, 0.0, 63.0)
        z1c = jnp.clip(z0 + 1.0, 0.0, 63.0)
        ya0 = y0c - 8.0
        ya1 = y1c - 8.0
        xa0 = x0c - 10.0
        xa1 = x1c - 10.0
        za0 = z0c - 10.0
        za1 = z1c - 10.0
        uy0 = 121.0 - ya0 * ya0
        uy1 = 121.0 - ya1 * ya1
        rx0 = xa0 * xa0
        rx1 = xa1 * xa1
        rz0 = za0 * za0
        rz1 = za1 * za1
        dx = x1c - xf
        dy = y1c - yf
        dz = z1c - zf
        exdx = 1.0 - dx
        exdy = 1.0 - dy
        exdz = 1.0 - dz
        acc = None
        for uy, py in ((uy0, dy), (uy1, exdy)):
            for rx, px in ((rx0, dx), (rx1, exdx)):
                u = uy - rx
                pxy = py * px
                q = jnp.where(rz0 <= u, dz, 0.0) + jnp.where(rz1 <= u, exdz, 0.0)
                term = pxy * q
                acc = term if acc is None else acc + term
        out_ref[0, hh, 0] = acc
        out_ref[0, hh, 1] = acc
        out_ref[0, hh, 2] = acc


@jax.jit
def kernel(x, W_def, b_def):
    B, H, W, D, C = x.shape
    xt = jnp.transpose(x, (0, 1, 4, 2, 3)).astype(jnp.bfloat16)
    b0, b1, b2 = b_def[0], b_def[1], b_def[2]
    z_lin = jnp.linspace(-1.0, 1.0, D)
    x_lin = jnp.linspace(-1.0, 1.0, W)
    y_lin = jnp.linspace(-1.0, 1.0, H)
    gx32 = jnp.broadcast_to(32.0 * x_lin[:, None], (W, D))
    hx = b0 * gx32 + 32.0
    gz32 = jnp.broadcast_to(32.0 * z_lin[None, :], (W, D))
    hz = b2 * gz32 + 32.0
    consts = jnp.stack([gx32, hx, gz32, hz], axis=0).astype(jnp.float32)
    sy = 16.0 * y_lin
    cy = b1 * sy + 16.0
    sycy = jnp.stack([sy, cy], axis=0).astype(jnp.float32)
    wu = jax.lax.bitcast_convert_type(W_def, jnp.uint32)
    wu = (wu + jnp.uint32(0x7FFF) + ((wu >> 16) & jnp.uint32(1))) & jnp.uint32(0xFFFF0000)
    Wr = jax.lax.bitcast_convert_type(wu, jnp.float32)

    out5 = pl.pallas_call(
        _body,
        grid=(B, H // _HB),
        in_specs=[
            pl.BlockSpec((1, _HB, C, W, D), lambda b, h: (b, h, 0, 0, 0)),
            pl.BlockSpec((4, W, D), lambda b, h: (0, 0, 0)),
            pl.BlockSpec(memory_space=pltpu.SMEM),
            pl.BlockSpec(memory_space=pltpu.SMEM),
        ],
        out_specs=pl.BlockSpec((1, _HB, C, W, D), lambda b, h: (b, h, 0, 0, 0)),
        out_shape=jax.ShapeDtypeStruct((B, H, C, W, D), jnp.float32),
        compiler_params=pltpu.CompilerParams(
            dimension_semantics=("parallel", "parallel"),
        ),
    )(xt, consts, sycy, Wr)
    return jnp.transpose(out5, (0, 1, 3, 4, 2))


# speedup vs baseline: 82.6154x; 1.2320x over previous
import jax
import jax.numpy as jnp
from jax.experimental import pallas as pl
from jax.experimental.pallas import tpu as pltpu

_HB = 16


def _body(x_ref, c_ref, sy_ref, w_ref, out_ref):
    gx32 = c_ref[0]
    hx = c_ref[1]
    gz32 = c_ref[2]
    hz = c_ref[3]
    h_base = pl.program_id(1) * _HB
    for hh in range(_HB):
        xu = jax.lax.bitcast_convert_type(x_ref[0, hh], jnp.uint32)
        xu = (xu + jnp.uint32(0x7FFF) + ((xu >> 16) & jnp.uint32(1))) \
            & jnp.uint32(0xFFFF0000)
        X = jax.lax.bitcast_convert_type(xu, jnp.float32)
        xc0 = X[0]
        xc1 = X[1]
        xc2 = X[2]
        t0 = xc0 * w_ref[0, 0] + xc1 * w_ref[1, 0] + xc2 * w_ref[2, 0]
        t1 = xc0 * w_ref[0, 1] + xc1 * w_ref[1, 1] + xc2 * w_ref[2, 1]
        t2 = xc0 * w_ref[0, 2] + xc1 * w_ref[1, 2] + xc2 * w_ref[2, 2]
        sy = sy_ref[0, h_base + hh]
        cy = sy_ref[1, h_base + hh]
        xf = t0 * gx32 + hx
        yf = t1 * sy + cy
        zf = t2 * gz32 + hz
        x0 = jnp.floor(xf)
        y0 = jnp.floor(yf)
        z0 = jnp.floor(zf)
        x0c = jnp.clip(x0, 0.0, 63.0)
        x1c = jnp.clip(x0 + 1.0, 0.0, 63.0)
        y0c = jnp.clip(y0, 0.0, 31.0)
        y1c = jnp.clip(y0 + 1.0, 0.0, 31.0)
        z0c = jnp.clip(z0, 0.0, 63.0)
        z1c = jnp.clip(z0 + 1.0, 0.0, 63.0)
        ya0 = y0c - 8.0
        ya1 = y1c - 8.0
        xa0 = x0c - 10.0
        xa1 = x1c - 10.0
        za0 = z0c - 10.0
        za1 = z1c - 10.0
        uy0 = 121.0 - ya0 * ya0
        uy1 = 121.0 - ya1 * ya1
        rx0 = xa0 * xa0
        rx1 = xa1 * xa1
        rz0 = za0 * za0
        rz1 = za1 * za1
        dx = x1c - xf
        dy = y1c - yf
        dz = z1c - zf
        exdx = 1.0 - dx
        exdy = 1.0 - dy
        exdz = 1.0 - dz
        acc = None
        for uy, py in ((uy0, dy), (uy1, exdy)):
            for rx, px in ((rx0, dx), (rx1, exdx)):
                u = uy - rx
                pxy = py * px
                q = jnp.where(rz0 <= u, dz, 0.0) + jnp.where(rz1 <= u, exdz, 0.0)
                term = pxy * q
                acc = term if acc is None else acc + term
        out_ref[0, hh, 0] = acc
        out_ref[0, hh, 1] = acc
        out_ref[0, hh, 2] = acc


@jax.jit
def kernel(x, W_def, b_def):
    B, H, W, D, C = x.shape
    xt = jnp.transpose(x, (0, 1, 4, 2, 3))
    b0, b1, b2 = b_def[0], b_def[1], b_def[2]
    z_lin = jnp.linspace(-1.0, 1.0, D)
    x_lin = jnp.linspace(-1.0, 1.0, W)
    y_lin = jnp.linspace(-1.0, 1.0, H)
    gx32 = jnp.broadcast_to(32.0 * x_lin[:, None], (W, D))
    hx = b0 * gx32 + 32.0
    gz32 = jnp.broadcast_to(32.0 * z_lin[None, :], (W, D))
    hz = b2 * gz32 + 32.0
    consts = jnp.stack([gx32, hx, gz32, hz], axis=0).astype(jnp.float32)
    sy = 16.0 * y_lin
    cy = b1 * sy + 16.0
    sycy = jnp.stack([sy, cy], axis=0).astype(jnp.float32)
    wu = jax.lax.bitcast_convert_type(W_def, jnp.uint32)
    wu = (wu + jnp.uint32(0x7FFF) + ((wu >> 16) & jnp.uint32(1))) & jnp.uint32(0xFFFF0000)
    Wr = jax.lax.bitcast_convert_type(wu, jnp.float32)

    out5 = pl.pallas_call(
        _body,
        grid=(B, H // _HB),
        in_specs=[
            pl.BlockSpec((1, _HB, C, W, D), lambda b, h: (b, h, 0, 0, 0)),
            pl.BlockSpec((4, W, D), lambda b, h: (0, 0, 0)),
            pl.BlockSpec(memory_space=pltpu.SMEM),
            pl.BlockSpec(memory_space=pltpu.SMEM),
        ],
        out_specs=pl.BlockSpec((1, _HB, C, W, D), lambda b, h: (b, h, 0, 0, 0)),
        out_shape=jax.ShapeDtypeStruct((B, H, C, W, D), jnp.float32),
        compiler_params=pltpu.CompilerParams(
            dimension_semantics=("parallel", "parallel"),
        ),
    )(xt, consts, sycy, Wr)
    return jnp.transpose(out5, (0, 1, 3, 4, 2))


# speedup vs baseline: 87.7708x; 1.0624x over previous
import jax
import jax.numpy as jnp
from jax.experimental import pallas as pl
from jax.experimental.pallas import tpu as pltpu

_HB = 32


def _body(x_ref, c_ref, sy_ref, w_ref, out_ref):
    gx32 = c_ref[0]
    hx = c_ref[1]
    gz32 = c_ref[2]
    hz = c_ref[3]
    h_base = pl.program_id(1) * _HB
    for hh in range(_HB):
        xu = jax.lax.bitcast_convert_type(x_ref[0, hh], jnp.uint32)
        xu = (xu + jnp.uint32(0x7FFF) + ((xu >> 16) & jnp.uint32(1))) \
            & jnp.uint32(0xFFFF0000)
        X = jax.lax.bitcast_convert_type(xu, jnp.float32)
        xc0 = X[0]
        xc1 = X[1]
        xc2 = X[2]
        t0 = xc0 * w_ref[0, 0] + xc1 * w_ref[1, 0] + xc2 * w_ref[2, 0]
        t1 = xc0 * w_ref[0, 1] + xc1 * w_ref[1, 1] + xc2 * w_ref[2, 1]
        t2 = xc0 * w_ref[0, 2] + xc1 * w_ref[1, 2] + xc2 * w_ref[2, 2]
        sy = sy_ref[0, h_base + hh]
        cy = sy_ref[1, h_base + hh]
        xf = t0 * gx32 + hx
        yf = t1 * sy + cy
        zf = t2 * gz32 + hz
        x0 = jnp.floor(xf)
        y0 = jnp.floor(yf)
        z0 = jnp.floor(zf)
        x0c = jnp.clip(x0, 0.0, 63.0)
        x1c = jnp.clip(x0 + 1.0, 0.0, 63.0)
        y0c = jnp.clip(y0, 0.0, 31.0)
        y1c = jnp.clip(y0 + 1.0, 0.0, 31.0)
        z0c = jnp.clip(z0, 0.0, 63.0)
        z1c = jnp.clip(z0 + 1.0, 0.0, 63.0)
        ya0 = y0c - 8.0
        ya1 = y1c - 8.0
        xa0 = x0c - 10.0
        xa1 = x1c - 10.0
        za0 = z0c - 10.0
        za1 = z1c - 10.0
        uy0 = 121.0 - ya0 * ya0
        uy1 = 121.0 - ya1 * ya1
        rx0 = xa0 * xa0
        rx1 = xa1 * xa1
        rz0 = za0 * za0
        rz1 = za1 * za1
        dx = x1c - xf
        dy = y1c - yf
        dz = z1c - zf
        exdx = 1.0 - dx
        exdy = 1.0 - dy
        exdz = 1.0 - dz
        acc = None
        for uy, py in ((uy0, dy), (uy1, exdy)):
            for rx, px in ((rx0, dx), (rx1, exdx)):
                u = uy - rx
                pxy = py * px
                q = jnp.where(rz0 <= u, dz, 0.0) + jnp.where(rz1 <= u, exdz, 0.0)
                term = pxy * q
                acc = term if acc is None else acc + term
        out_ref[0, hh, 0] = acc
        out_ref[0, hh, 1] = acc
        out_ref[0, hh, 2] = acc


@jax.jit
def kernel(x, W_def, b_def):
    B, H, W, D, C = x.shape
    xt = jnp.transpose(x, (0, 1, 4, 2, 3))
    b0, b1, b2 = b_def[0], b_def[1], b_def[2]
    z_lin = jnp.linspace(-1.0, 1.0, D)
    x_lin = jnp.linspace(-1.0, 1.0, W)
    y_lin = jnp.linspace(-1.0, 1.0, H)
    gx32 = jnp.broadcast_to(32.0 * x_lin[:, None], (W, D))
    hx = b0 * gx32 + 32.0
    gz32 = jnp.broadcast_to(32.0 * z_lin[None, :], (W, D))
    hz = b2 * gz32 + 32.0
    consts = jnp.stack([gx32, hx, gz32, hz], axis=0).astype(jnp.float32)
    sy = 16.0 * y_lin
    cy = b1 * sy + 16.0
    sycy = jnp.stack([sy, cy], axis=0).astype(jnp.float32)
    wu = jax.lax.bitcast_convert_type(W_def, jnp.uint32)
    wu = (wu + jnp.uint32(0x7FFF) + ((wu >> 16) & jnp.uint32(1))) & jnp.uint32(0xFFFF0000)
    Wr = jax.lax.bitcast_convert_type(wu, jnp.float32)

    out5 = pl.pallas_call(
        _body,
        grid=(B, H // _HB),
        in_specs=[
            pl.BlockSpec((1, _HB, C, W, D), lambda b, h: (b, h, 0, 0, 0)),
            pl.BlockSpec((4, W, D), lambda b, h: (0, 0, 0)),
            pl.BlockSpec(memory_space=pltpu.SMEM),
            pl.BlockSpec(memory_space=pltpu.SMEM),
        ],
        out_specs=pl.BlockSpec((1, _HB, C, W, D), lambda b, h: (b, h, 0, 0, 0)),
        out_shape=jax.ShapeDtypeStruct((B, H, C, W, D), jnp.float32),
        compiler_params=pltpu.CompilerParams(
            dimension_semantics=("parallel", "parallel"),
        ),
    )(xt, consts, sycy, Wr)
    return jnp.transpose(out5, (0, 1, 3, 4, 2))


# speedup vs baseline: 104.3227x; 1.1886x over previous
import jax
import jax.numpy as jnp
from jax.experimental import pallas as pl
from jax.experimental.pallas import tpu as pltpu

_HB = 32


def _body(x_ref, c_ref, sy_ref, w_ref, out_ref):
    gx32 = c_ref[0]
    hx = c_ref[1]
    gz32 = c_ref[2]
    hz = c_ref[3]
    lanemask = jax.lax.broadcasted_iota(jnp.int32, (64, 128), 1) < 64
    h_base = pl.program_id(1) * _HB
    for hh in range(_HB // 2):
        Xp = jnp.concatenate(
            [x_ref[0, 2 * hh], x_ref[0, 2 * hh + 1]], axis=-1)
        xu = jax.lax.bitcast_convert_type(Xp, jnp.uint32)
        xu = (xu + jnp.uint32(0x7FFF) + ((xu >> 16) & jnp.uint32(1))) \
            & jnp.uint32(0xFFFF0000)
        X = jax.lax.bitcast_convert_type(xu, jnp.float32)
        xc0 = X[0]
        xc1 = X[1]
        xc2 = X[2]
        t0 = xc0 * w_ref[0, 0] + xc1 * w_ref[1, 0] + xc2 * w_ref[2, 0]
        t1 = xc0 * w_ref[0, 1] + xc1 * w_ref[1, 1] + xc2 * w_ref[2, 1]
        t2 = xc0 * w_ref[0, 2] + xc1 * w_ref[1, 2] + xc2 * w_ref[2, 2]
        sy_a = sy_ref[0, h_base + 2 * hh]
        sy_b = sy_ref[0, h_base + 2 * hh + 1]
        cy_a = sy_ref[1, h_base + 2 * hh]
        cy_b = sy_ref[1, h_base + 2 * hh + 1]
        sy = jnp.where(lanemask, sy_a, sy_b)
        cy = jnp.where(lanemask, cy_a, cy_b)
        xf = t0 * gx32 + hx
        yf = t1 * sy + cy
        zf = t2 * gz32 + hz
        x0 = jnp.floor(xf)
        y0 = jnp.floor(yf)
        z0 = jnp.floor(zf)
        x0c = jnp.clip(x0, 0.0, 63.0)
        x1c = jnp.clip(x0 + 1.0, 0.0, 63.0)
        y0c = jnp.clip(y0, 0.0, 31.0)
        y1c = jnp.clip(y0 + 1.0, 0.0, 31.0)
        z0c = jnp.clip(z0, 0.0, 63.0)
        z1c = jnp.clip(z0 + 1.0, 0.0, 63.0)
        ya0 = y0c - 8.0
        ya1 = y1c - 8.0
        xa0 = x0c - 10.0
        xa1 = x1c - 10.0
        za0 = z0c - 10.0
        za1 = z1c - 10.0
        uy0 = 121.0 - ya0 * ya0
        uy1 = 121.0 - ya1 * ya1
        rx0 = xa0 * xa0
        rx1 = xa1 * xa1
        rz0 = za0 * za0
        rz1 = za1 * za1
        dx = x1c - xf
        dy = y1c - yf
        dz = z1c - zf
        exdx = 1.0 - dx
        exdy = 1.0 - dy
        exdz = 1.0 - dz
        acc = None
        for uy, py in ((uy0, dy), (uy1, exdy)):
            for rx, px in ((rx0, dx), (rx1, exdx)):
                u = uy - rx
                pxy = py * px
                q = jnp.where(rz0 <= u, dz, 0.0) + jnp.where(rz1 <= u, exdz, 0.0)
                term = pxy * q
                acc = term if acc is None else acc + term
        acc_a = acc[:, 0:64]
        acc_b = acc[:, 64:128]
        out_ref[0, 2 * hh, 0] = acc_a
        out_ref[0, 2 * hh, 1] = acc_a
        out_ref[0, 2 * hh, 2] = acc_a
        out_ref[0, 2 * hh + 1, 0] = acc_b
        out_ref[0, 2 * hh + 1, 1] = acc_b
        out_ref[0, 2 * hh + 1, 2] = acc_b


@jax.jit
def kernel(x, W_def, b_def):
    B, H, W, D, C = x.shape
    xt = jnp.transpose(x, (0, 1, 4, 2, 3))
    b0, b1, b2 = b_def[0], b_def[1], b_def[2]
    z_lin = jnp.linspace(-1.0, 1.0, D)
    x_lin = jnp.linspace(-1.0, 1.0, W)
    y_lin = jnp.linspace(-1.0, 1.0, H)
    gx32 = jnp.broadcast_to(32.0 * x_lin[:, None], (W, 2 * D))
    hx = b0 * gx32 + 32.0
    gz32 = jnp.broadcast_to(jnp.tile(32.0 * z_lin, 2)[None, :], (W, 2 * D))
    hz = b2 * gz32 + 32.0
    consts = jnp.stack([gx32, hx, gz32, hz], axis=0).astype(jnp.float32)
    sy = 16.0 * y_lin
    cy = b1 * sy + 16.0
    sycy = jnp.stack([sy, cy], axis=0).astype(jnp.float32)
    wu = jax.lax.bitcast_convert_type(W_def, jnp.uint32)
    wu = (wu + jnp.uint32(0x7FFF) + ((wu >> 16) & jnp.uint32(1))) & jnp.uint32(0xFFFF0000)
    Wr = jax.lax.bitcast_convert_type(wu, jnp.float32)

    out5 = pl.pallas_call(
        _body,
        grid=(B, H // _HB),
        in_specs=[
            pl.BlockSpec((1, _HB, C, W, D), lambda b, h: (b, h, 0, 0, 0)),
            pl.BlockSpec((4, W, 2 * D), lambda b, h: (0, 0, 0)),
            pl.BlockSpec(memory_space=pltpu.SMEM),
            pl.BlockSpec(memory_space=pltpu.SMEM),
        ],
        out_specs=pl.BlockSpec((1, _HB, C, W, D), lambda b, h: (b, h, 0, 0, 0)),
        out_shape=jax.ShapeDtypeStruct((B, H, C, W, D), jnp.float32),
        compiler_params=pltpu.CompilerParams(
            dimension_semantics=("parallel", "parallel"),
        ),
    )(xt, consts, sycy, Wr)
    return jnp.transpose(out5, (0, 1, 3, 4, 2))
